# baseline (speedup 1.0000x reference)
import jax
import jax.numpy as jnp
from jax import lax
from jax.experimental import pallas as pl
from jax.experimental.pallas import tpu as pltpu
from jax.experimental.pallas import tpu_sc as plsc

NC = 2
NS = 16
L = 16
NW = NC * NS
B = 16384
D = 64
BPW = B // NW
CH = 128
NCH = BPW // CH
NG = CH // L


def _sqrt16(x):
    bits = plsc.bitcast(x, jnp.int32)
    y = plsc.bitcast(jnp.int32(0x1FBD1DF5) + (bits >> 1), jnp.float32)
    for _ in range(3):
        y = 0.5 * (y + x / y)
    return y


def _sc_body(h2d, r2d, t2d, ent, rel, out,
             hidx, ridx, tidx, hidx2, ridx2, tidx2,
             hv0, rv0, tv0, hv1, rv1, tv1, sc2,
             sh0, sr0, st0, sh1, sr1, st1):
    c = lax.axis_index("c")
    s = lax.axis_index("s")
    wid = s * NC + c

    pltpu.sync_copy(h2d.at[pl.ds(NCH * wid, NCH)], hidx)
    pltpu.sync_copy(r2d.at[pl.ds(NCH * wid, NCH)], ridx)
    pltpu.sync_copy(t2d.at[pl.ds(NCH * wid, NCH)], tidx)

    for src, dst in ((hidx, hidx2), (ridx, ridx2), (tidx, tidx2)):
        for j in range(NCH):
            for v in range(CH // L):
                dst[j, pl.ds(v * L, L)] = src[j, pl.ds(v * L, L)] >> 1

    hv = (hv0, hv1)
    rv = (rv0, rv1)
    tv = (tv0, tv1)
    sems = ((sh0, sr0, st0), (sh1, sr1, st1))

    def fire(j):
        slot = j % 2
        return (
            pltpu.async_copy(ent.at[hidx2.at[j]], hv[slot], sems[slot][0]),
            pltpu.async_copy(rel.at[ridx2.at[j]], rv[slot], sems[slot][1]),
            pltpu.async_copy(ent.at[tidx2.at[j]], tv[slot], sems[slot][2]),
        )

    inflight = [fire(0), fire(1)]

    iota = lax.iota(jnp.int32, L)

    for j in range(NCH):
        slot = j % 2
        for cp in inflight[j]:
            cp.wait()

        def group_body(g, carry, j=j, slot=slot):
            row16 = g * L + iota
            hid = hidx[j, pl.ds(g * L, L)]
            rid = ridx[j, pl.ds(g * L, L)]
            tid = tidx[j, pl.ds(g * L, L)]
            hcol0 = (hid & 1) * D
            rcol0 = (rid & 1) * D
            tcol0 = (tid & 1) * D

            def dim_body(d, acc):
                hh = plsc.load_gather(hv[slot], [row16, hcol0 + d])
                re = plsc.load_gather(rv[slot], [row16, rcol0 + d])
                tt = plsc.load_gather(tv[slot], [row16, tcol0 + d])
                df = (hh + re) - tt
                return acc + df * df

            acc = lax.fori_loop(0, D, dim_body, jnp.zeros((L,), jnp.float32),
                                unroll=8)
            sc2[j, pl.ds(g * L, L)] = _sqrt16(acc)
            return carry

        lax.fori_loop(0, NG, group_body, 0)

        if j + 2 < NCH:
            inflight.append(fire(j + 2))

    pltpu.sync_copy(sc2, out.at[pl.ds(NCH * wid, NCH)])


@jax.jit
def kernel(head_ids, relation_ids, tail_ids, entity_table, relation_table):
    h2d = head_ids.astype(jnp.int32).reshape(NW * NCH, CH)
    r2d = relation_ids.astype(jnp.int32).reshape(NW * NCH, CH)
    t2d = tail_ids.astype(jnp.int32).reshape(NW * NCH, CH)
    ent = entity_table.reshape(-1, 2 * D)
    rel = relation_table.reshape(-1, 2 * D)

    mesh = plsc.VectorSubcoreMesh(core_axis_name="c", subcore_axis_name="s")
    scratch = [
        pltpu.VMEM((NCH, CH), jnp.int32),
        pltpu.VMEM((NCH, CH), jnp.int32),
        pltpu.VMEM((NCH, CH), jnp.int32),
        pltpu.VMEM((NCH, CH), jnp.int32),
        pltpu.VMEM((NCH, CH), jnp.int32),
        pltpu.VMEM((NCH, CH), jnp.int32),
        pltpu.VMEM((CH, 2 * D), jnp.float32),
        pltpu.VMEM((CH, 2 * D), jnp.float32),
        pltpu.VMEM((CH, 2 * D), jnp.float32),
        pltpu.VMEM((CH, 2 * D), jnp.float32),
        pltpu.VMEM((CH, 2 * D), jnp.float32),
        pltpu.VMEM((CH, 2 * D), jnp.float32),
        pltpu.VMEM((NCH, CH), jnp.float32),
    ] + [pltpu.SemaphoreType.DMA] * 6

    run = pl.kernel(
        _sc_body,
        out_type=jax.ShapeDtypeStruct((NW * NCH, CH), jnp.float32),
        mesh=mesh,
        scratch_types=scratch,
        compiler_params=pltpu.CompilerParams(
            needs_layout_passes=False, use_tc_tiling_on_sc=True),
    )
    out = run(h2d, r2d, t2d, ent, rel)
    return out.reshape(B)


# speedup vs baseline: 1.7545x; 1.7545x over previous
import jax
import jax.numpy as jnp
from jax import lax
from jax.experimental import pallas as pl
from jax.experimental.pallas import tpu as pltpu
from jax.experimental.pallas import tpu_sc as plsc

NC = 2
NS = 16
L = 16
NW = NC * NS
B = 16384
D = 64
BPW = B // NW
CH = 128
NCH = BPW // CH
NG = CH // L


def _sqrt16(x):
    bits = plsc.bitcast(x, jnp.int32)
    y = plsc.bitcast(jnp.int32(0x1FBD1DF5) + (bits >> 1), jnp.float32)
    for _ in range(3):
        y = 0.5 * (y + x / y)
    return y


def _sc_body(h2d, r2d, t2d, ent, rel, out,
             hidx, ridx, tidx,
             hv0, rv0, tv0, hv1, rv1, tv1, ps, sc2,
             sh0, sr0, st0, sh1, sr1, st1):
    c = lax.axis_index("c")
    s = lax.axis_index("s")
    wid = s * NC + c

    pltpu.sync_copy(h2d.at[pl.ds(NCH * wid, NCH)], hidx)
    pltpu.sync_copy(r2d.at[pl.ds(NCH * wid, NCH)], ridx)
    pltpu.sync_copy(t2d.at[pl.ds(NCH * wid, NCH)], tidx)

    hv = (hv0, hv1)
    rv = (rv0, rv1)
    tv = (tv0, tv1)
    sems = ((sh0, sr0, st0), (sh1, sr1, st1))

    iota = lax.iota(jnp.int32, L)

    def fire(j):
        slot = j % 2

        def group_dma(g, carry, j=j, slot=slot):
            hvec = hidx[j, pl.ds(g * L, L)]
            rvec = ridx[j, pl.ds(g * L, L)]
            tvec = tidx[j, pl.ds(g * L, L)]
            for lane in range(L):
                m = iota == lane
                hid = lax.reduce_max(jnp.where(m, hvec, -1), axes=(0,))
                rid = lax.reduce_max(jnp.where(m, rvec, -1), axes=(0,))
                tid = lax.reduce_max(jnp.where(m, tvec, -1), axes=(0,))
                r = g * L + lane
                pltpu.make_async_copy(
                    ent.at[pl.ds(hid, 1)], hv[slot].at[pl.ds(r, 1)],
                    sems[slot][0]).start()
                pltpu.make_async_copy(
                    rel.at[pl.ds(rid, 1)], rv[slot].at[pl.ds(r, 1)],
                    sems[slot][1]).start()
                pltpu.make_async_copy(
                    ent.at[pl.ds(tid, 1)], tv[slot].at[pl.ds(r, 1)],
                    sems[slot][2]).start()
            return carry

        lax.fori_loop(0, NG, group_dma, 0)

    def drain(j):
        slot = j % 2
        pltpu.make_async_copy(
            ent.at[pl.ds(0, CH)], hv[slot], sems[slot][0]).wait()
        pltpu.make_async_copy(
            ent.at[pl.ds(0, CH)], rv[slot], sems[slot][1]).wait()
        pltpu.make_async_copy(
            ent.at[pl.ds(0, CH)], tv[slot], sems[slot][2]).wait()

    fire(0)
    fire(1)

    for j in range(NCH):
        slot = j % 2
        drain(j)

        def row_body(r, carry, slot=slot):
            acc = None
            for k in range(D // L):
                hh = hv[slot][r, pl.ds(k * L, L)]
                re = rv[slot][r, pl.ds(k * L, L)]
                tt = tv[slot][r, pl.ds(k * L, L)]
                df = (hh + re) - tt
                acc = df * df if acc is None else acc + df * df
            ps[r] = acc
            return carry

        lax.fori_loop(0, CH, row_body, 0, unroll=4)

        for g in range(NG):
            rows16 = g * L + iota
            acc = jnp.zeros((L,), jnp.float32)
            for k in range(L):
                col = jnp.full((L,), k, jnp.int32)
                acc = acc + plsc.load_gather(ps, [rows16, col])
            sc2[j, pl.ds(g * L, L)] = _sqrt16(acc)

        if j + 2 < NCH:
            fire(j + 2)

    pltpu.sync_copy(sc2, out.at[pl.ds(NCH * wid, NCH)])


@jax.jit
def kernel(head_ids, relation_ids, tail_ids, entity_table, relation_table):
    h2d = head_ids.astype(jnp.int32).reshape(NW * NCH, CH)
    r2d = relation_ids.astype(jnp.int32).reshape(NW * NCH, CH)
    t2d = tail_ids.astype(jnp.int32).reshape(NW * NCH, CH)

    mesh = plsc.VectorSubcoreMesh(core_axis_name="c", subcore_axis_name="s")
    scratch = [
        pltpu.VMEM((NCH, CH), jnp.int32),
        pltpu.VMEM((NCH, CH), jnp.int32),
        pltpu.VMEM((NCH, CH), jnp.int32),
        pltpu.VMEM((CH, D), jnp.float32),
        pltpu.VMEM((CH, D), jnp.float32),
        pltpu.VMEM((CH, D), jnp.float32),
        pltpu.VMEM((CH, D), jnp.float32),
        pltpu.VMEM((CH, D), jnp.float32),
        pltpu.VMEM((CH, D), jnp.float32),
        pltpu.VMEM((CH, L), jnp.float32),
        pltpu.VMEM((NCH, CH), jnp.float32),
    ] + [pltpu.SemaphoreType.DMA] * 6

    run = pl.kernel(
        _sc_body,
        out_type=jax.ShapeDtypeStruct((NW * NCH, CH), jnp.float32),
        mesh=mesh,
        scratch_types=scratch,
        compiler_params=pltpu.CompilerParams(
            needs_layout_passes=False, use_tc_tiling_on_sc=True),
    )
    out = run(h2d, r2d, t2d, entity_table, relation_table)
    return out.reshape(B)
